# baseline (speedup 1.0000x reference)
import functools

import jax
import jax.numpy as jnp
from jax import lax
from jax.experimental import pallas as pl
from jax.experimental.pallas import tpu as pltpu
from jax.experimental.pallas import tpu_sc as plsc

D = 64
LANES = 16
GSUB = 100


def _make_body(n_batch, seq, n_cores, n_subcores):
  n_workers = n_cores * n_subcores
  assert n_batch % n_workers == 0
  assert seq % (2 * GSUB) == 0
  b_per_w = n_batch // n_workers
  n_it = b_per_w
  assert n_it % 2 == 0
  g_per_it = seq // GSUB
  idx_rows_w = b_per_w * seq // GSUB

  mesh = plsc.VectorSubcoreMesh(core_axis_name="c", subcore_axis_name="s")

  @functools.partial(
      pl.kernel,
      out_type=jax.ShapeDtypeStruct((n_batch, seq // 2, 2 * D), jnp.float32),
      mesh=mesh,
      compiler_params=pltpu.CompilerParams(use_tc_tiling_on_sc=False),
      scratch_types=[
          pltpu.VMEM((idx_rows_w, GSUB), jnp.int32),
          pltpu.VMEM((seq, D), jnp.float32),
          pltpu.VMEM((seq, D), jnp.float32),
          pltpu.VMEM((seq // 2, 2 * D), jnp.float32),
          pltpu.VMEM((seq // 2, 2 * D), jnp.float32),
          pltpu.VMEM((seq, D), jnp.float32),
          pltpu.SemaphoreType.DMA,
          pltpu.SemaphoreType.DMA,
          pltpu.SemaphoreType.DMA,
          pltpu.SemaphoreType.DMA,
      ],
  )
  def body(idx_hbm, table_hbm, pos_hbm, out_hbm,
           idx_v, gath0, gath1, wr0, wr1, pe_v,
           sem_g0, sem_g1, sem_w0, sem_w1):
    gath = (gath0, gath1)
    wr = (wr0, wr1)
    sem_g = (sem_g0, sem_g1)
    sem_w = (sem_w0, sem_w1)

    wid = lax.axis_index("s") * n_cores + lax.axis_index("c")
    batch0 = wid * b_per_w
    idx_row0 = wid * idx_rows_w

    pltpu.sync_copy(idx_hbm.at[pl.ds(idx_row0, idx_rows_w)], idx_v)
    pltpu.sync_copy(pos_hbm.at[pl.ds(0, seq)], pe_v)

    def fire_gathers(t, b):
      for j in range(g_per_it):
        src = table_hbm.at[idx_v.at[t * g_per_it + j]]
        pltpu.async_copy(src, gath[b].at[pl.ds(j * GSUB, GSUB)], sem_g[b])

    def drain_gathers(t, b):
      for j in range(g_per_it):
        src = table_hbm.at[idx_v.at[t * g_per_it + j]]
        pltpu.make_async_copy(src, gath[b].at[pl.ds(j * GSUB, GSUB)],
                              sem_g[b]).wait()

    def fire_write(t, b):
      pltpu.async_copy(wr[b], out_hbm.at[batch0 + t], sem_w[b])

    def drain_write(t, b):
      pltpu.make_async_copy(wr[b], out_hbm.at[batch0 + t], sem_w[b]).wait()

    def add_pe_pack(b):
      def pack_body(r, carry):
        half = r >> 1
        col0 = (r & 1) * D
        for q in range(D // LANES):
          sl = pl.ds(q * LANES, LANES)
          v = gath[b][r, sl] + pe_v[r, sl]
          wr[b][half, pl.ds(col0 + q * LANES, LANES)] = v
        return carry
      lax.fori_loop(0, seq, pack_body, None, unroll=False)

    fire_gathers(0, 0)

    def step(t2, carry):
      t = 2 * t2

      @pl.when(t2 > 0)
      def _():
        drain_write(t - 1, 1)
      fire_gathers(t + 1, 1)
      drain_gathers(t, 0)
      add_pe_pack(0)
      fire_write(t, 0)

      t = 2 * t2 + 1
      drain_write(t - 1, 0)

      @pl.when(t2 < n_it // 2 - 1)
      def _():
        fire_gathers(t + 1, 0)
      drain_gathers(t, 1)
      add_pe_pack(1)
      fire_write(t, 1)
      return carry

    lax.fori_loop(0, n_it // 2, step, None, unroll=False)
    drain_write(n_it - 1, 1)

  return body


def kernel(inputs, table, pos_encoding):
  b, s = inputs.shape
  info = plsc.get_sparse_core_info()
  idx2d = inputs.reshape(b * s // GSUB, GSUB).astype(jnp.int32)
  body = _make_body(b, s, info.num_cores, info.num_subcores)
  out = body(idx2d, table, pos_encoding)
  return out.reshape(b, s, D)
